# baseline (speedup 1.0000x reference)
import numpy as np
import jax
import jax.numpy as jnp
from jax.experimental import pallas as pl
from jax.experimental.pallas import tpu as pltpu

_N, _L, _K = 128, 8192, 20
_C = 2560
_RT = 8
_R = _L // 128
_J = _R // _RT
_G = _C // _K
_GT = _G // 20


def _threefry_bits(x1):
    k1 = jnp.uint32(42)
    k2 = jnp.uint32(0 ^ 42 ^ 0x1BD11BDA)
    ks = (jnp.uint32(0), k1, k2)
    rot = ((13, 15, 26, 6), (17, 29, 16, 24))
    x1 = x1 + k1
    x0 = x1
    x1 = ((x1 << 13) | (x1 >> 19)) ^ x0
    for i in range(5):
        rs = rot[i % 2][1:] if i == 0 else rot[i % 2]
        for r in rs:
            x0 = x0 + x1
            x1 = ((x1 << r) | (x1 >> (32 - r))) ^ x0
        x0 = x0 + ks[(i + 1) % 3]
        x1 = x1 + ks[(i + 2) % 3] + jnp.uint32(i + 1)
    return x0 ^ x1


def _body(dt_ref, lin_ref, pg_ref, pw_ref, e1_ref, e1t_ref,
          ct_ref, vc_ref, xt_ref, mk_ref, x_out, c_out):
    n = pl.program_id(0)
    j = pl.program_id(1)
    ct = ct_ref[0]
    vc = vc_ref[0]
    dtn = dt_ref[n]
    c_new = ct + vc * dtn
    probs = jnp.clip(c_new, 0.0, 1.0) + 1e-8
    v = jnp.log(probs)

    base = (n * _R + j * _RT) * _C
    flat = lin_ref[0] + base
    bits = _threefry_bits(flat.astype(jnp.uint32))
    fb = (bits >> 9) | jnp.uint32(0x3F800000)
    floats = jax.lax.bitcast_convert_type(fb, jnp.float32) - 1.0
    tiny = jnp.float32(np.finfo(np.float32).tiny)
    u = jnp.maximum(tiny, floats + tiny)
    v = v + (-jnp.log(-jnp.log(u)))

    pg = pg_ref[0]
    neg_inf = jnp.float32(-np.inf)
    m = v
    for s in (1, 2, 4, 8, 16):
        cand = pltpu.roll(m, _C - s, 1)
        m = jnp.maximum(m, jnp.where(pg + s < _K, cand, neg_inf))

    e1 = e1_ref[0]
    e1t = e1t_ref[0]
    mc = jnp.dot(jnp.where(pg == 0, m, 0.0), e1,
                 preferred_element_type=jnp.float32)
    mb = jnp.dot(mc, e1t, preferred_element_type=jnp.float32)
    contrib = jnp.where(v == mb, pw_ref[0], 0.0)
    s2 = jnp.dot(contrib, e1, preferred_element_type=jnp.float32)
    xs = 127 - (jax.lax.bitcast_convert_type(s2, jnp.int32) >> 23)

    mk = mk_ref[0]
    x_out[0] = jnp.where(mk != 0, xs, xt_ref[0])

    mke = jnp.dot(mk.astype(jnp.float32), e1t,
                  preferred_element_type=jnp.float32)
    c_out[0] = jnp.where(mke > 0.5, c_new, ct)


def kernel(xt, ct, vc_t, dt, mask):
    ct3 = ct.reshape(_N, _R, _C)
    vc3 = vc_t.reshape(_N, _R, _C)
    xt3 = xt.reshape(_N, _R, 128)
    mk3 = mask.astype(jnp.int32).reshape(_N, _R, 128)

    lin = np.arange(_RT * _C, dtype=np.int32).reshape(1, _RT, _C)
    pg_np = lin % _K
    pw_np = np.exp2(-pg_np).astype(np.float32)
    cg = np.arange(_C, dtype=np.int32) // _K
    e1_np = (cg[:, None] == np.arange(_G)[None, :]).astype(np.float32)
    e1_np = e1_np.reshape(1, _C, _G)
    e1t_np = e1_np.reshape(1, _C, _G).transpose(0, 2, 1).copy()

    const_spec = lambda shp: pl.BlockSpec(shp, lambda n, j: (0, 0, 0))
    x_new, c_new = pl.pallas_call(
        _body,
        grid=(_N, _J),
        in_specs=[
            pl.BlockSpec(memory_space=pltpu.SMEM),
            const_spec((1, _RT, _C)),
            const_spec((1, _RT, _C)),
            const_spec((1, _RT, _C)),
            const_spec((1, _C, _G)),
            const_spec((1, _G, _C)),
            pl.BlockSpec((1, _RT, _C), lambda n, j: (n, j, 0)),
            pl.BlockSpec((1, _RT, _C), lambda n, j: (n, j, 0)),
            pl.BlockSpec((1, _RT, 128), lambda n, j: (n, j, 0)),
            pl.BlockSpec((1, _RT, 128), lambda n, j: (n, j, 0)),
        ],
        out_specs=[
            pl.BlockSpec((1, _RT, 128), lambda n, j: (n, j, 0)),
            pl.BlockSpec((1, _RT, _C), lambda n, j: (n, j, 0)),
        ],
        out_shape=[
            jax.ShapeDtypeStruct((_N, _R, 128), jnp.int32),
            jax.ShapeDtypeStruct((_N, _R, _C), jnp.float32),
        ],
    )(dt, jnp.asarray(lin), jnp.asarray(pg_np), jnp.asarray(pw_np),
      jnp.asarray(e1_np), jnp.asarray(e1t_np),
      ct3, vc3, xt3, mk3)
    return x_new.reshape(_N, _L), c_new.reshape(_N, _L, _K)
